# baseline (speedup 1.0000x reference)
import jax
import jax.numpy as jnp
from jax import lax
from jax.experimental import pallas as pl
from jax.experimental.pallas import tpu as pltpu
from jax.experimental.pallas import tpu_sc as plsc

GAMMA = 12.0
PI = 3.141592653589793

_NC, _NS, _L = 2, 16, 16
_NW = _NC * _NS


def _sc_gather(ent_hbm, rel_hbm, hi_hbm, ri_hbm, ti_hbm,
               h_out, r_out, t_out,
               hidx_v, ridx_v, tidx_v, sem):
    wid = lax.axis_index("s") * _NC + lax.axis_index("c")
    bw = hidx_v.shape[0]
    base = wid * bw
    nchunk = bw // _L
    pltpu.sync_copy(hi_hbm.at[wid], hidx_v)
    pltpu.sync_copy(ri_hbm.at[wid], ridx_v)
    pltpu.sync_copy(ti_hbm.at[wid], tidx_v)

    tables = ((ent_hbm, hidx_v, h_out),
              (rel_hbm, ridx_v, r_out),
              (ent_hbm, tidx_v, t_out))

    def chunk(c, carry):
        off = pl.multiple_of(c * _L, _L)
        for tbl, idx_v, out in tables:
            vec = idx_v[pl.ds(off, _L)]
            for k in range(_L):
                pltpu.async_copy(tbl.at[vec[k]], out.at[base + off + k], sem)
        @pl.when(c > 0)
        def _():
            prev = base + (c - 1) * _L
            for tbl, _idx, out in tables:
                pltpu.make_async_copy(
                    tbl.at[pl.ds(0, _L)], out.at[pl.ds(prev, _L)], sem
                ).wait()
        return carry

    lax.fori_loop(0, nchunk, chunk, 0)
    last = base + (nchunk - 1) * _L
    for tbl, _idx, out in tables:
        pltpu.make_async_copy(
            tbl.at[pl.ds(0, _L)], out.at[pl.ds(last, _L)], sem
        ).wait()


def _gather_all(entity_emb, relation_emb, h_idx, r_idx, t_idx):
    B = h_idx.shape[0]
    D = entity_emb.shape[1]
    bw = B // _NW
    mesh = plsc.VectorSubcoreMesh(core_axis_name="c", subcore_axis_name="s")
    f32 = jnp.float32
    kern = pl.kernel(
        _sc_gather,
        out_type=(
            jax.ShapeDtypeStruct((B, D), f32),
            jax.ShapeDtypeStruct((B, D), f32),
            jax.ShapeDtypeStruct((B, D), f32),
        ),
        mesh=mesh,
        scratch_types=[
            pltpu.VMEM((bw,), jnp.int32),
            pltpu.VMEM((bw,), jnp.int32),
            pltpu.VMEM((bw,), jnp.int32),
            pltpu.SemaphoreType.DMA,
        ],
    )
    hi = h_idx.reshape(_NW, bw)
    ri = r_idx.reshape(_NW, bw)
    ti = t_idx.reshape(_NW, bw)
    return kern(entity_emb, relation_emb, hi, ri, ti)


_SS_ROWS = 8000


def _sumsq_body(x_ref, o_ref, acc_ref):
    i = pl.program_id(0)

    @pl.when(i == 0)
    def _():
        acc_ref[0, 0] = 0.0

    x = x_ref[...]
    acc_ref[0, 0] += jnp.sum(x * x)

    @pl.when(i == pl.num_programs(0) - 1)
    def _():
        o_ref[0, 0] = acc_ref[0, 0]


def _sumsq(entity_emb):
    N, D = entity_emb.shape
    rows = _SS_ROWS if N % _SS_ROWS == 0 else D
    grid = N // rows
    return pl.pallas_call(
        _sumsq_body,
        grid=(grid,),
        in_specs=[pl.BlockSpec((rows, D), lambda i: (i, 0))],
        out_specs=pl.BlockSpec(
            (1, 1), lambda i: (0, 0), memory_space=pltpu.SMEM),
        out_shape=jax.ShapeDtypeStruct((1, 1), jnp.float32),
        scratch_shapes=[pltpu.SMEM((1, 1), jnp.float32)],
    )(entity_emb)


_SCORE_ROWS = 2048


def _score_body(ss_ref, h_ref, r_ref, t_ref, o_ref):
    half = h_ref.shape[1] // 2
    norm = jnp.sqrt(ss_ref[0, 0])
    scale = PI / (norm + 1e-9)
    phase = r_ref[:, :half] * scale
    r_re = jnp.cos(phase)
    r_im = jnp.sin(phase)
    h_r = h_ref[:, :half]
    h_i = h_ref[:, half:]
    rot_r = h_r * r_re - h_i * r_im
    rot_i = h_r * r_im + h_i * r_re
    re_d = rot_r - t_ref[:, :half]
    im_d = rot_i - t_ref[:, half:]
    dist = jnp.sum(jnp.sqrt(re_d * re_d + im_d * im_d + 1e-9), axis=1)
    o_ref[...] = (GAMMA - dist)[:, None]


def _score(sumsq, h, r, t):
    B, D = h.shape
    rows = _SCORE_ROWS
    grid = B // rows
    out = pl.pallas_call(
        _score_body,
        grid=(grid,),
        in_specs=[
            pl.BlockSpec((1, 1), lambda i: (0, 0), memory_space=pltpu.SMEM),
            pl.BlockSpec((rows, D), lambda i: (i, 0)),
            pl.BlockSpec((rows, D), lambda i: (i, 0)),
            pl.BlockSpec((rows, D), lambda i: (i, 0)),
        ],
        out_specs=pl.BlockSpec((rows, 1), lambda i: (i, 0)),
        out_shape=jax.ShapeDtypeStruct((B, 1), jnp.float32),
    )(sumsq, h, r, t)
    return out[:, 0]


def kernel(entity_emb, relation_emb, h_idx, r_idx, t_idx):
    h, r, t = _gather_all(entity_emb, relation_emb,
                          h_idx.astype(jnp.int32),
                          r_idx.astype(jnp.int32),
                          t_idx.astype(jnp.int32))
    ss = _sumsq(entity_emb)
    return _score(ss, h, r, t)


# speedup vs baseline: 1.0031x; 1.0031x over previous
import jax
import jax.numpy as jnp
from jax import lax
from jax.experimental import pallas as pl
from jax.experimental.pallas import tpu as pltpu
from jax.experimental.pallas import tpu_sc as plsc

GAMMA = 12.0
PI = 3.141592653589793

_NC, _NS, _L = 2, 16, 16
_NW = _NC * _NS


def _sc_gather(ent_hbm, rel_hbm, hi_hbm, ri_hbm, ti_hbm,
               h_out, r_out, t_out,
               hidx_v, ridx_v, tidx_v, sem):
    wid = lax.axis_index("s") * _NC + lax.axis_index("c")
    bw = hidx_v.shape[0]
    base = wid * bw
    nchunk = bw // _L
    pltpu.sync_copy(hi_hbm.at[wid], hidx_v)
    pltpu.sync_copy(ri_hbm.at[wid], ridx_v)
    pltpu.sync_copy(ti_hbm.at[wid], tidx_v)

    tables = ((ent_hbm, hidx_v, h_out),
              (rel_hbm, ridx_v, r_out),
              (ent_hbm, tidx_v, t_out))

    _LAG = 4

    def chunk(c, carry):
        off = pl.multiple_of(c * _L, _L)
        for tbl, idx_v, out in tables:
            vec = idx_v[pl.ds(off, _L)]
            for k in range(_L):
                pltpu.async_copy(tbl.at[vec[k]], out.at[base + off + k], sem)
        @pl.when(c >= _LAG)
        def _():
            prev = base + (c - _LAG) * _L
            for tbl, _idx, out in tables:
                pltpu.make_async_copy(
                    tbl.at[pl.ds(0, _L)], out.at[pl.ds(prev, _L)], sem
                ).wait()
        return carry

    lax.fori_loop(0, nchunk, chunk, 0)
    for c in range(nchunk - _LAG, nchunk):
        tail = base + c * _L
        for tbl, _idx, out in tables:
            pltpu.make_async_copy(
                tbl.at[pl.ds(0, _L)], out.at[pl.ds(tail, _L)], sem
            ).wait()


def _gather_all(entity_emb, relation_emb, h_idx, r_idx, t_idx):
    B = h_idx.shape[0]
    D = entity_emb.shape[1]
    bw = B // _NW
    mesh = plsc.VectorSubcoreMesh(core_axis_name="c", subcore_axis_name="s")
    f32 = jnp.float32
    kern = pl.kernel(
        _sc_gather,
        out_type=(
            jax.ShapeDtypeStruct((B, D), f32),
            jax.ShapeDtypeStruct((B, D), f32),
            jax.ShapeDtypeStruct((B, D), f32),
        ),
        mesh=mesh,
        scratch_types=[
            pltpu.VMEM((bw,), jnp.int32),
            pltpu.VMEM((bw,), jnp.int32),
            pltpu.VMEM((bw,), jnp.int32),
            pltpu.SemaphoreType.DMA,
        ],
    )
    hi = h_idx.reshape(_NW, bw)
    ri = r_idx.reshape(_NW, bw)
    ti = t_idx.reshape(_NW, bw)
    return kern(entity_emb, relation_emb, hi, ri, ti)


_SS_ROWS = 8000


def _sumsq_body(x_ref, o_ref, acc_ref):
    i = pl.program_id(0)

    @pl.when(i == 0)
    def _():
        acc_ref[0, 0] = 0.0

    x = x_ref[...]
    acc_ref[0, 0] += jnp.sum(x * x)

    @pl.when(i == pl.num_programs(0) - 1)
    def _():
        o_ref[0, 0] = acc_ref[0, 0]


def _sumsq(entity_emb):
    N, D = entity_emb.shape
    rows = _SS_ROWS if N % _SS_ROWS == 0 else D
    grid = N // rows
    return pl.pallas_call(
        _sumsq_body,
        grid=(grid,),
        in_specs=[pl.BlockSpec((rows, D), lambda i: (i, 0))],
        out_specs=pl.BlockSpec(
            (1, 1), lambda i: (0, 0), memory_space=pltpu.SMEM),
        out_shape=jax.ShapeDtypeStruct((1, 1), jnp.float32),
        scratch_shapes=[pltpu.SMEM((1, 1), jnp.float32)],
    )(entity_emb)


_SCORE_ROWS = 2048


def _score_body(ss_ref, h_ref, r_ref, t_ref, o_ref):
    half = h_ref.shape[1] // 2
    norm = jnp.sqrt(ss_ref[0, 0])
    scale = PI / (norm + 1e-9)
    phase = r_ref[:, :half] * scale
    r_re = jnp.cos(phase)
    r_im = jnp.sin(phase)
    h_r = h_ref[:, :half]
    h_i = h_ref[:, half:]
    rot_r = h_r * r_re - h_i * r_im
    rot_i = h_r * r_im + h_i * r_re
    re_d = rot_r - t_ref[:, :half]
    im_d = rot_i - t_ref[:, half:]
    dist = jnp.sum(jnp.sqrt(re_d * re_d + im_d * im_d + 1e-9), axis=1)
    o_ref[...] = (GAMMA - dist)[:, None]


def _score(sumsq, h, r, t):
    B, D = h.shape
    rows = _SCORE_ROWS
    grid = B // rows
    out = pl.pallas_call(
        _score_body,
        grid=(grid,),
        in_specs=[
            pl.BlockSpec((1, 1), lambda i: (0, 0), memory_space=pltpu.SMEM),
            pl.BlockSpec((rows, D), lambda i: (i, 0)),
            pl.BlockSpec((rows, D), lambda i: (i, 0)),
            pl.BlockSpec((rows, D), lambda i: (i, 0)),
        ],
        out_specs=pl.BlockSpec((rows, 1), lambda i: (i, 0)),
        out_shape=jax.ShapeDtypeStruct((B, 1), jnp.float32),
    )(sumsq, h, r, t)
    return out[:, 0]


def kernel(entity_emb, relation_emb, h_idx, r_idx, t_idx):
    h, r, t = _gather_all(entity_emb, relation_emb,
                          h_idx.astype(jnp.int32),
                          r_idx.astype(jnp.int32),
                          t_idx.astype(jnp.int32))
    ss = _sumsq(entity_emb)
    return _score(ss, h, r, t)


# speedup vs baseline: 1.4350x; 1.4306x over previous
import jax
import jax.numpy as jnp
from jax import lax
from jax.experimental import pallas as pl
from jax.experimental.pallas import tpu as pltpu
from jax.experimental.pallas import tpu_sc as plsc

GAMMA = 12.0
PI = 3.141592653589793

_NC, _NS, _L = 2, 16, 16
_NW = _NC * _NS

_NE = 1000000
_HALF_E = _NE // 2
_CH = 80
_NCHG = _HALF_E // _CH
_PERW = _NCHG // _NW
_EXTRA = _NCHG - _PERW * _NW

_NRR = 1000
_HALF_R = _NRR // 2
_RPAD = 512


def _pack_kernel(ent_hbm, rel_hbm, packed_out, relp_out, ss_out,
                 in0, in1, out0, out1, acc_v, sem_f, sem_w):
    wid = lax.axis_index("s") * _NC + lax.axis_index("c")
    D = ent_hbm.shape[1]
    nh = D // _L

    def fire_fetch(g, inb):
        p0 = pl.multiple_of(g * _CH, 16)
        p1 = pl.multiple_of(_HALF_E + g * _CH, 16)
        pltpu.async_copy(ent_hbm.at[pl.ds(p0, _CH)],
                         inb.at[pl.ds(0, _CH)], sem_f)
        pltpu.async_copy(ent_hbm.at[pl.ds(p1, _CH)],
                         inb.at[pl.ds(_CH, _CH)], sem_f)

    def drain_fetch(inb):
        pltpu.make_async_copy(ent_hbm.at[pl.ds(0, 2 * _CH)], inb,
                              sem_f).wait()

    def fire_write(g, outb):
        p0 = pl.multiple_of(g * _CH, 16)
        pltpu.async_copy(outb, packed_out.at[pl.ds(p0, _CH)], sem_w)

    def drain_write(outb):
        pltpu.make_async_copy(packed_out.at[pl.ds(0, _CH)], outb,
                              sem_w).wait()

    def repack(inb, outb, accs):
        def row_body(r, a):
            lo = [inb[r, pl.ds(k * _L, _L)] for k in range(nh)]
            hi = [inb[_CH + r, pl.ds(k * _L, _L)] for k in range(nh)]
            for k in range(nh):
                outb[r, pl.ds(k * _L, _L)] = lo[k]
                outb[r, pl.ds(D + k * _L, _L)] = hi[k]
            a = tuple(a[k] + lo[k] * lo[k] for k in range(nh)) \
                + tuple(a[nh + k] + hi[k] * hi[k] for k in range(nh))
            return a
        return lax.fori_loop(0, _CH, row_body, accs)

    def gchunk(i):
        return wid + _NW * i

    def slot(c, inb, oinb, outb, accs, last):
        drain_fetch(inb)

        @pl.when(c + 1 < last)
        def _():
            fire_fetch(gchunk(c + 1), oinb)

        @pl.when(c >= 2)
        def _():
            drain_write(outb)

        accs = repack(inb, outb, accs)
        fire_write(gchunk(c), outb)
        return accs

    fire_fetch(gchunk(0), in0)
    accs0 = (jnp.zeros((_L,), jnp.float32),) * (2 * nh)

    def pair(i, accs):
        accs = slot(2 * i, in0, in1, out0, accs, _PERW)
        accs = slot(2 * i + 1, in1, in0, out1, accs, _PERW)
        return accs

    accs = lax.fori_loop(0, _PERW // 2, pair, accs0)
    c = _PERW - 1
    drain_fetch(in0)

    @pl.when(wid < _EXTRA)
    def _():
        fire_fetch(_NW * _PERW + wid, in1)

    drain_write(out0)
    accs = repack(in0, out0, accs)
    fire_write(gchunk(c), out0)

    has_extra = wid < _EXTRA

    @pl.when(has_extra)
    def _():
        drain_fetch(in1)

    drain_write(out1)
    eaccs = repack(in1, out1, accs)
    accs = tuple(jnp.where(has_extra, a2, a1) for a1, a2 in zip(accs, eaccs))

    @pl.when(has_extra)
    def _():
        fire_write(_NW * _PERW + wid, out1)

    @pl.when(wid == _NW - 1)
    def _():
        for r0, sz, szu in ((0, 80, 88), (80, 80, 88), (160, 80, 88),
                            (240, 80, 88), (320, 80, 88), (400, 80, 88),
                            (480, 24, 24)):
            pltpu.async_copy(rel_hbm.at[pl.ds(r0, sz)],
                             in0.at[pl.ds(0, sz)], sem_f)
            pltpu.async_copy(rel_hbm.at[pl.ds(496 + r0, szu)],
                             in1.at[pl.ds(0, szu)], sem_f)
            pltpu.make_async_copy(rel_hbm.at[pl.ds(0, sz)],
                                  in0.at[pl.ds(0, sz)], sem_f).wait()
            pltpu.make_async_copy(rel_hbm.at[pl.ds(0, szu)],
                                  in1.at[pl.ds(0, szu)], sem_f).wait()

            def rrow(r, carry):
                for k in range(nh):
                    out0[r, pl.ds(k * _L, _L)] = in0[r, pl.ds(k * _L, _L)]
                    out0[r, pl.ds(D + k * _L, _L)] = \
                        in1[4 + r, pl.ds(k * _L, _L)]
                return carry

            lax.fori_loop(0, sz, rrow, 0)
            pltpu.sync_copy(out0.at[pl.ds(0, sz)],
                            relp_out.at[pl.ds(r0, sz)])

    total = accs[0]
    for k in range(1, 2 * nh):
        total = total + accs[k]
    acc_v[0, pl.ds(0, _L)] = total
    pltpu.sync_copy(acc_v, ss_out.at[wid])
    drain_write(out0)

    @pl.when(has_extra)
    def _():
        drain_write(out1)


def _pack(entity_emb, relation_emb):
    N, D = entity_emb.shape
    mesh = plsc.VectorSubcoreMesh(core_axis_name="c", subcore_axis_name="s")
    f32 = jnp.float32
    kern = pl.kernel(
        _pack_kernel,
        out_type=(
            jax.ShapeDtypeStruct((N // 2, 2 * D), f32),
            jax.ShapeDtypeStruct((_RPAD, 2 * D), f32),
            jax.ShapeDtypeStruct((_NW, 1, _L), f32),
        ),
        mesh=mesh,
        compiler_params=pltpu.CompilerParams(needs_layout_passes=False),
        scratch_types=[
            pltpu.VMEM((2 * _CH, D), f32),
            pltpu.VMEM((2 * _CH, D), f32),
            pltpu.VMEM((_CH, 2 * D), f32),
            pltpu.VMEM((_CH, 2 * D), f32),
            pltpu.VMEM((1, _L), f32),
            pltpu.SemaphoreType.DMA,
            pltpu.SemaphoreType.DMA,
        ],
    )
    return kern(entity_emb, relation_emb)


_GCH = 128


def _gather_kernel(packed, relp, hi_hbm, ri_hbm, ti_hbm,
                   h_out, r_out, t_out,
                   hidx_v, ridx_v, tidx_v, pidx_v, fbuf, obuf, sem):
    wid = lax.axis_index("s") * _NC + lax.axis_index("c")
    bw = hidx_v.shape[1]
    base = pl.multiple_of(wid * bw, 512)
    nchunk = bw // _GCH
    D = packed.shape[1] // 2
    pltpu.sync_copy(hi_hbm.at[wid], hidx_v)
    pltpu.sync_copy(ri_hbm.at[wid], ridx_v)
    pltpu.sync_copy(ti_hbm.at[wid], tidx_v)

    for tbl, idx_v, out, half in ((packed, hidx_v, h_out, _HALF_E),
                                  (relp, ridx_v, r_out, _HALF_R),
                                  (packed, tidx_v, t_out, _HALF_E)):
        def do_chunk(c, carry, tbl=tbl, idx_v=idx_v, half=half):
            off = pl.multiple_of(c * _GCH, _GCH)
            for g in range(_GCH // _L):
                v = idx_v[0, pl.ds(off + g * _L, _L)]
                pidx_v[0, pl.ds(g * _L, _L)] = jnp.where(v >= half,
                                                         v - half, v)
            pltpu.async_copy(tbl.at[pidx_v.at[0]], fbuf, sem)
            pltpu.make_async_copy(tbl.at[pl.ds(0, _GCH)], fbuf, sem).wait()
            hvs = []
            for g in range(_GCH // _L):
                v = idx_v[0, pl.ds(off + g * _L, _L)]
                hvs.append(jnp.where(v >= half, D, 0))

            def dim_body(d, carry2):
                for g in range(_GCH // _L):
                    pos = lax.iota(jnp.int32, _L) + g * _L
                    vals = plsc.load_gather(fbuf, [pos, hvs[g] + d])
                    obuf[d, pl.ds(off + g * _L, _L)] = vals
                return carry2

            lax.fori_loop(0, D, dim_body, 0)
            return carry

        lax.fori_loop(0, nchunk, do_chunk, 0)
        pltpu.sync_copy(obuf, out.at[:, pl.ds(base, bw)])


def _gather(packed, relp, h_idx, r_idx, t_idx):
    B = h_idx.shape[0]
    bw = B // _NW
    D = packed.shape[1] // 2
    mesh = plsc.VectorSubcoreMesh(core_axis_name="c", subcore_axis_name="s")
    f32 = jnp.float32
    kern = pl.kernel(
        _gather_kernel,
        out_type=(
            jax.ShapeDtypeStruct((D, B), f32),
            jax.ShapeDtypeStruct((D, B), f32),
            jax.ShapeDtypeStruct((D, B), f32),
        ),
        mesh=mesh,
        compiler_params=pltpu.CompilerParams(needs_layout_passes=False),
        scratch_types=[
            pltpu.VMEM((1, bw), jnp.int32),
            pltpu.VMEM((1, bw), jnp.int32),
            pltpu.VMEM((1, bw), jnp.int32),
            pltpu.VMEM((1, _GCH), jnp.int32),
            pltpu.VMEM((_GCH, 2 * D), f32),
            pltpu.VMEM((D, bw), f32),
            pltpu.SemaphoreType.DMA,
        ],
    )
    hi = h_idx.reshape(_NW, 1, bw)
    ri = r_idx.reshape(_NW, 1, bw)
    ti = t_idx.reshape(_NW, 1, bw)
    return kern(packed, relp, hi, ri, ti)


_SCORE_COLS = 2048


def _score_body(scp_ref, h_ref, r_ref, t_ref, o_ref):
    half = h_ref.shape[0] // 2
    ss = jnp.sum(scp_ref[...])
    norm = jnp.sqrt(ss)
    scale = PI / (norm + 1e-9)
    phase = r_ref[:half, :] * scale
    r_re = jnp.cos(phase)
    r_im = jnp.sin(phase)
    h_r = h_ref[:half, :]
    h_i = h_ref[half:, :]
    rot_r = h_r * r_re - h_i * r_im
    rot_i = h_r * r_im + h_i * r_re
    re_d = rot_r - t_ref[:half, :]
    im_d = rot_i - t_ref[half:, :]
    dist = jnp.sum(jnp.sqrt(re_d * re_d + im_d * im_d + 1e-9), axis=0)
    o_ref[...] = (GAMMA - dist)[None, :]


def _score(ss_parts, h_t, r_t, t_t):
    D, B = h_t.shape
    cols = _SCORE_COLS
    grid = B // cols
    out = pl.pallas_call(
        _score_body,
        grid=(grid,),
        in_specs=[
            pl.BlockSpec((_NW, _L), lambda i: (0, 0)),
            pl.BlockSpec((D, cols), lambda i: (0, i)),
            pl.BlockSpec((D, cols), lambda i: (0, i)),
            pl.BlockSpec((D, cols), lambda i: (0, i)),
        ],
        out_specs=pl.BlockSpec((1, cols), lambda i: (0, i)),
        out_shape=jax.ShapeDtypeStruct((1, B), jnp.float32),
    )(ss_parts, h_t, r_t, t_t)
    return out[0]


def kernel(entity_emb, relation_emb, h_idx, r_idx, t_idx):
    packed, relp, ss = _pack(entity_emb, relation_emb)
    h_t, r_t, t_t = _gather(packed, relp,
                            h_idx.astype(jnp.int32),
                            r_idx.astype(jnp.int32),
                            t_idx.astype(jnp.int32))
    return _score(ss.reshape(_NW, _L), h_t, r_t, t_t)
